# baseline (speedup 1.0000x reference)
import functools

import jax
import jax.numpy as jnp
from jax import lax
from jax.experimental import pallas as pl
from jax.experimental.pallas import tpu as pltpu
from jax.experimental.pallas import tpu_sc as plsc

N = 10000
E = 320000
D = 128
DEPTH = 10

NC = 2
NS = 16
HALF = D // NC
HW = HALF // 2
NP = 10240
RPT = NP // NS
C = 128
EPT = 20480
NCHUNK = EPT // C
NWB = RPT // C
OUTROWS = NC * (DEPTH + 1) * NP
NBUF = 4


def _body(ein, xr, peh, zz, out, hb, edata,
          ering, colx, gb16, prod, hbuf, dinv_l, ones_b, alph,
          tmp_r, tmp_c, gsems, esems, acc, degacc):
    c = lax.axis_index("c")
    s = lax.axis_index("s")
    base0 = c * ((DEPTH + 1) * NP)
    hbase = c * NP
    e0 = s * NCHUNK
    r0 = s * RPT

    pltpu.sync_copy(peh, alph)
    for g in range(2):
        sl = pl.ds(16 * g, 16)
        pe = alph[sl]
        alph[sl] = 1.0 - 2.0 / (jnp.exp(pe * 2.0) + 1.0)

    one16 = jnp.ones((16,), jnp.float32)
    for g in range(C // 16):
        ones_b[pl.ds(16 * g, 16)] = one16

    for k in range(RPT // C):
        pltpu.sync_copy(zz, acc.at[pl.ds(r0 + k * C, C)])
    for k in range(RPT // HALF):
        pltpu.sync_copy(zz.at[0], degacc.at[pl.ds(r0 + k * HALF, HALF)])
    plsc.subcore_barrier()

    def deg_chunk(j, carry):
        pltpu.sync_copy(ein.at[e0 + j], ering.at[0])
        pltpu.sync_copy(ones_b, degacc.at[ering.at[0, 0]], add=True)
        return carry
    lax.fori_loop(0, NCHUNK, deg_chunk, 0)
    plsc.subcore_barrier()

    pltpu.sync_copy(degacc.at[pl.ds(r0, RPT)], dinv_l)
    for g in range(RPT // 16):
        sl = pl.ds(16 * g, 16)
        d = dinv_l[sl]
        d = jnp.where(d < 0.5, d + 1.0, d)
        i = lax.bitcast_convert_type(d, jnp.int32)
        i = 0x5F3759DF - lax.shift_right_logical(i, 1)
        y = lax.bitcast_convert_type(i, jnp.float32)
        for _ in range(3):
            y = y * (1.5 - 0.5 * d * y * y)
        dinv_l[sl] = y
    pltpu.sync_copy(dinv_l, degacc.at[pl.ds(r0, RPT)])
    plsc.subcore_barrier()

    def val_chunk(j, carry):
        pltpu.sync_copy(ein.at[e0 + j], ering.at[0])
        pltpu.sync_copy(degacc.at[ering.at[0, 0]], tmp_r)
        pltpu.sync_copy(degacc.at[ering.at[0, 1]], tmp_c)
        for g in range(C // 16):
            sl = pl.ds(16 * g, 16)
            at = lax.bitcast_convert_type(ering[0, 2, sl], jnp.float32)
            v = tmp_r[sl] * at * tmp_c[sl]
            ering[0, 2, sl] = lax.bitcast_convert_type(v, jnp.int32)
        pltpu.sync_copy(ering.at[0], edata.at[e0 + j])
        return carry
    lax.fori_loop(0, NCHUNK, val_chunk, 0)

    def pack_rows(sbuf, dst):
        def prow(r, cy):
            for g in range(HALF // 32):
                a = sbuf[r, pl.ds(32 * g, 16)]
                b = sbuf[r, pl.ds(32 * g + 16, 16)]
                ta = lax.bitcast_convert_type(a, jnp.int32) + 0x8000
                tb = lax.bitcast_convert_type(b, jnp.int32) + 0x8000
                ra = lax.shift_right_logical(ta, 16)
                rb = lax.bitwise_and(tb, -65536)
                dst[r, pl.ds(16 * g, 16)] = lax.bitwise_or(ra, rb)
            return cy
        lax.fori_loop(0, C, prow, 0)

    for k in range(NWB):
        pltpu.sync_copy(xr.at[pl.ds(c * NP + r0 + k * C, C)], prod)
        pltpu.sync_copy(prod, out.at[pl.ds(base0 + r0 + k * C, C)])
        pack_rows(prod, hbuf)
        pltpu.sync_copy(hbuf, hb.at[pl.ds(hbase + r0 + k * C, C)])
    plsc.subcore_barrier()

    hb16 = jnp.full((16,), hbase, jnp.int32)

    def scale(b):
        def sgroup(g, cy):
            v16 = lax.bitcast_convert_type(ering[b, 2, pl.ds(16 * g, 16)],
                                           jnp.float32)
            for i in range(16):
                e = 16 * g + i
                vv = jnp.full((16,), v16[i], jnp.float32)
                for f in range(HALF // 32):
                    w = gb16[b, e, pl.ds(16 * f, 16)]
                    lo = lax.bitcast_convert_type(
                        lax.shift_left(w, 16), jnp.float32)
                    hi = lax.bitcast_convert_type(
                        lax.bitwise_and(w, -65536), jnp.float32)
                    prod[e, pl.ds(32 * f, 16)] = lo * vv
                    prod[e, pl.ds(32 * f + 16, 16)] = hi * vv
            return cy
        lax.fori_loop(0, C // 16, sgroup, 0)

    def estream(jj, b):
        pltpu.async_copy(edata.at[e0 + jj], ering.at[b], esems.at[b])

    def ewait(jj, b):
        pltpu.make_async_copy(edata.at[e0 + jj], ering.at[b],
                              esems.at[b]).wait()

    def gissue(jj, b):
        for g in range(C // 16):
            sl = pl.ds(16 * g, 16)
            colx[b, sl] = ering[b, 1, sl] + hb16
        pltpu.async_copy(hb.at[colx.at[b]], gb16.at[b], gsems.at[b])

    def gwait(jj, b):
        pltpu.make_async_copy(hb.at[colx.at[b]], gb16.at[b],
                              gsems.at[b]).wait()

    def scatter(b):
        pltpu.sync_copy(prod, acc.at[ering.at[b, 0]], add=True)

    def depth_body(l, carry):
        for b in range(NBUF):
            estream(b, b)
        for b in range(NBUF - 1):
            ewait(b, b)
            gissue(b, b)

        def slot(jj, b):
            bp = (b + NBUF - 1) % NBUF
            gwait(jj, b)
            ewait(jj + NBUF - 1, bp)
            gissue(jj + NBUF - 1, bp)
            scale(b)
            scatter(b)
            estream(jj + NBUF, b)

        def pipe(t, cy):
            for b in range(NBUF):
                slot(NBUF * t + b, b)
            return cy
        lax.fori_loop(0, (NCHUNK - NBUF) // NBUF, pipe, 0)
        for jj in range(NCHUNK - NBUF - (NCHUNK - NBUF) % NBUF,
                        NCHUNK - NBUF + 1):
            slot(jj, jj % NBUF)
        for jj in range(NCHUNK - NBUF + 1, NCHUNK):
            b = jj % NBUF
            gwait(jj, b)
            scale(b)
            scatter(b)
        ewait(NCHUNK, (NCHUNK - NBUF) % NBUF)
        plsc.subcore_barrier()

        a16 = alph[pl.ds(l - 1, 16)]
        av = jnp.full((16,), a16[0], jnp.float32)

        def wb_chunk(k, cy):
            rr = r0 + k * C
            pltpu.sync_copy(acc.at[pl.ds(rr, C)], prod)
            pltpu.sync_copy(zz, acc.at[pl.ds(rr, C)])
            def wrow(r, cy):
                for f in range(HALF // 16):
                    sl = pl.ds(16 * f, 16)
                    prod[r, sl] = prod[r, sl] * av
                return cy
            lax.fori_loop(0, C, wrow, 0)
            pltpu.sync_copy(prod, out.at[pl.ds(base0 + l * NP + rr, C)])
            pack_rows(prod, hbuf)
            pltpu.sync_copy(hbuf, hb.at[pl.ds(hbase + rr, C)])
            return cy
        lax.fori_loop(0, NWB, wb_chunk, 0)
        plsc.subcore_barrier()
        return carry
    lax.fori_loop(1, DEPTH + 1, depth_body, 0)


_mesh = plsc.VectorSubcoreMesh(core_axis_name="c", subcore_axis_name="s")

_sc_call = functools.partial(
    pl.kernel,
    out_type=(jax.ShapeDtypeStruct((OUTROWS, HALF), jnp.float32),
              jax.ShapeDtypeStruct((NC * NP, HW), jnp.int32),
              jax.ShapeDtypeStruct((NS * NCHUNK + 2 * NBUF, 3, C), jnp.int32)),
    mesh=_mesh,
    compiler_params=pltpu.CompilerParams(use_tc_tiling_on_sc=False),
    scratch_types=[
        pltpu.VMEM((NBUF, 3, C), jnp.int32),
        pltpu.VMEM((NBUF, C), jnp.int32),
        pltpu.VMEM((NBUF, C, HW), jnp.int32),
        pltpu.VMEM((C, HALF), jnp.float32),
        pltpu.VMEM((C, HW), jnp.int32),
        pltpu.VMEM((RPT,), jnp.float32),
        pltpu.VMEM((C,), jnp.float32),
        pltpu.VMEM((32,), jnp.float32),
        pltpu.VMEM((C,), jnp.float32),
        pltpu.VMEM((C,), jnp.float32),
        pltpu.SemaphoreType.DMA((NBUF,)),
        pltpu.SemaphoreType.DMA((NBUF,)),
        pltpu.VMEM_SHARED((NP, HALF), jnp.float32),
        pltpu.VMEM_SHARED((NP,), jnp.float32),
    ],
)(_body)


def kernel(x, edge_index, edge_attr, pe_alphas):
    row = edge_index[0]
    col = edge_index[1]
    pad = EPT - E // NS
    rp = jnp.concatenate(
        [row.reshape(NS, E // NS), jnp.full((NS, pad), N, jnp.int32)],
        axis=1).reshape(NS * NCHUNK, C)
    cp = jnp.concatenate(
        [col.reshape(NS, E // NS), jnp.zeros((NS, pad), jnp.int32)],
        axis=1).reshape(NS * NCHUNK, C)
    ap = lax.bitcast_convert_type(
        jnp.concatenate(
            [edge_attr.reshape(NS, E // NS),
             jnp.zeros((NS, pad), jnp.float32)], axis=1),
        jnp.int32).reshape(NS * NCHUNK, C)
    ein = jnp.stack([rp, cp, ap], axis=1)
    ein = jnp.pad(ein, ((0, 2 * NBUF), (0, 0), (0, 0)))
    xp = jnp.pad(x, ((0, NP - N), (0, 0)))
    xr = xp.reshape(NP, NC, HALF).transpose(1, 0, 2).reshape(NC * NP, HALF)
    pe = jnp.pad(pe_alphas.astype(jnp.float32), (0, 32 - DEPTH))
    zz = jnp.zeros((C, HALF), jnp.float32)

    out, _, _ = _sc_call(ein, xr, pe, zz)
    out = out.reshape(NC, DEPTH + 1, NP, HALF)[:, :, :N, :]
    out = out.transpose(2, 1, 0, 3).reshape(N, DEPTH + 1, D)
    return out


# speedup vs baseline: 1.7202x; 1.7202x over previous
import functools

import jax
import jax.numpy as jnp
from jax import lax
from jax.experimental import pallas as pl
from jax.experimental.pallas import tpu as pltpu
from jax.experimental.pallas import tpu_sc as plsc

N = 10000
E = 320000
D = 128
DEPTH = 10

NC = 2
NS = 16
HALF = D // NC
HW = HALF // 2
NP = 10240
RPT = NP // NS
C = 128
EPT = 20480
NCHUNK = EPT // C
NWB = RPT // C
OUTROWS = NC * (DEPTH + 1) * NP
NBUF = 2


def _body(xr, rowh, colh, attrh, peh, zz, out, hb,
          row_v, col_v, val_v, gb16, prod, hbuf, dinv_l, ones_b, alph,
          tmp_r, tmp_c, gsems, acc, degacc):
    c = lax.axis_index("c")
    s = lax.axis_index("s")
    base0 = c * ((DEPTH + 1) * NP)
    hbase = c * NP
    e0 = s * NCHUNK
    r0 = s * RPT

    pltpu.sync_copy(rowh.at[pl.ds(e0, NCHUNK)], row_v)
    pltpu.sync_copy(colh.at[pl.ds(e0, NCHUNK)], col_v)
    pltpu.sync_copy(attrh.at[pl.ds(e0, NCHUNK)], val_v)
    pltpu.sync_copy(peh, alph)

    for g in range(2):
        sl = pl.ds(16 * g, 16)
        pe = alph[sl]
        alph[sl] = 1.0 - 2.0 / (jnp.exp(pe * 2.0) + 1.0)

    one16 = jnp.ones((16,), jnp.float32)
    for g in range(C // 16):
        ones_b[pl.ds(16 * g, 16)] = one16

    for k in range(RPT // C):
        pltpu.sync_copy(zz, acc.at[pl.ds(r0 + k * C, C)])
    for k in range(RPT // HALF):
        pltpu.sync_copy(zz.at[0], degacc.at[pl.ds(r0 + k * HALF, HALF)])
    plsc.subcore_barrier()

    def deg_chunk(j, carry):
        pltpu.sync_copy(ones_b, degacc.at[row_v.at[j]], add=True)
        return carry
    lax.fori_loop(0, NCHUNK, deg_chunk, 0)
    plsc.subcore_barrier()

    pltpu.sync_copy(degacc.at[pl.ds(r0, RPT)], dinv_l)
    for g in range(RPT // 16):
        sl = pl.ds(16 * g, 16)
        d = dinv_l[sl]
        d = jnp.where(d < 0.5, d + 1.0, d)
        i = lax.bitcast_convert_type(d, jnp.int32)
        i = 0x5F3759DF - lax.shift_right_logical(i, 1)
        y = lax.bitcast_convert_type(i, jnp.float32)
        for _ in range(3):
            y = y * (1.5 - 0.5 * d * y * y)
        dinv_l[sl] = y
    pltpu.sync_copy(dinv_l, degacc.at[pl.ds(r0, RPT)])
    plsc.subcore_barrier()

    b016 = jnp.full((16,), hbase, jnp.int32)

    def val_chunk(j, carry):
        pltpu.sync_copy(degacc.at[row_v.at[j]], tmp_r)
        pltpu.sync_copy(degacc.at[col_v.at[j]], tmp_c)
        for g in range(C // 16):
            sl = pl.ds(16 * g, 16)
            val_v[j, sl] = tmp_r[sl] * val_v[j, sl] * tmp_c[sl]
            col_v[j, sl] = col_v[j, sl] + b016
        return carry
    lax.fori_loop(0, NCHUNK, val_chunk, 0)

    def pack_rows(sbuf, dst):
        def prow(r, cy):
            for g in range(HALF // 32):
                a = sbuf[r, pl.ds(32 * g, 16)]
                b = sbuf[r, pl.ds(32 * g + 16, 16)]
                ta = lax.bitcast_convert_type(a, jnp.int32) + 0x8000
                tb = lax.bitcast_convert_type(b, jnp.int32) + 0x8000
                ra = lax.shift_right_logical(ta, 16)
                rb = lax.bitwise_and(tb, -65536)
                dst[r, pl.ds(16 * g, 16)] = lax.bitwise_or(ra, rb)
            return cy
        lax.fori_loop(0, C, prow, 0)

    for k in range(NWB):
        pltpu.sync_copy(xr.at[pl.ds(c * NP + r0 + k * C, C)], prod)
        pltpu.sync_copy(prod, out.at[pl.ds(base0 + r0 + k * C, C)])
        pack_rows(prod, hbuf)
        pltpu.sync_copy(hbuf, hb.at[pl.ds(hbase + r0 + k * C, C)])
    plsc.subcore_barrier()

    def depth_body(l, carry):
        pltpu.async_copy(hb.at[col_v.at[0]], gb16.at[0], gsems.at[0])

        def chunk(j, cy):
            for b in range(NBUF):
                jj = j * NBUF + b
                gb = gb16.at[b]
                bp = (b + 1) % NBUF
                pltpu.make_async_copy(hb.at[col_v.at[jj]], gb,
                                      gsems.at[b]).wait()

                @pl.when(jj + 1 < NCHUNK)
                def _():
                    pltpu.async_copy(hb.at[col_v.at[jj + 1]],
                                     gb16.at[bp], gsems.at[bp])

                for g in range(C // 16):
                    v16 = val_v[jj, pl.ds(16 * g, 16)]
                    for i in range(16):
                        e = 16 * g + i
                        vv = jnp.full((16,), v16[i], jnp.float32)
                        for f in range(HALF // 32):
                            w = gb[e, pl.ds(16 * f, 16)]
                            lo = lax.bitcast_convert_type(
                                lax.shift_left(w, 16), jnp.float32)
                            hi = lax.bitcast_convert_type(
                                lax.bitwise_and(w, -65536), jnp.float32)
                            prod[e, pl.ds(32 * f, 16)] = lo * vv
                            prod[e, pl.ds(32 * f + 16, 16)] = hi * vv
                pltpu.sync_copy(prod, acc.at[row_v.at[jj]], add=True)
            return cy
        lax.fori_loop(0, NCHUNK // NBUF, chunk, 0)
        plsc.subcore_barrier()

        a16 = alph[pl.ds(l - 1, 16)]
        av = jnp.full((16,), a16[0], jnp.float32)

        def wb_chunk(k, cy):
            rr = r0 + k * C
            pltpu.sync_copy(acc.at[pl.ds(rr, C)], prod)
            pltpu.sync_copy(zz, acc.at[pl.ds(rr, C)])

            def wrow(r, cy2):
                for f in range(HALF // 16):
                    sl = pl.ds(16 * f, 16)
                    prod[r, sl] = prod[r, sl] * av
                return cy2
            lax.fori_loop(0, C, wrow, 0)
            pltpu.sync_copy(prod, out.at[pl.ds(base0 + l * NP + rr, C)])
            pack_rows(prod, hbuf)
            pltpu.sync_copy(hbuf, hb.at[pl.ds(hbase + rr, C)])
            return cy
        lax.fori_loop(0, NWB, wb_chunk, 0)
        plsc.subcore_barrier()
        return carry
    lax.fori_loop(1, DEPTH + 1, depth_body, 0)


_mesh = plsc.VectorSubcoreMesh(core_axis_name="c", subcore_axis_name="s")

_sc_call = functools.partial(
    pl.kernel,
    out_type=(jax.ShapeDtypeStruct((OUTROWS, HALF), jnp.float32),
              jax.ShapeDtypeStruct((NC * NP, HW), jnp.int32)),
    mesh=_mesh,
    compiler_params=pltpu.CompilerParams(use_tc_tiling_on_sc=False),
    scratch_types=[
        pltpu.VMEM((NCHUNK, C), jnp.int32),
        pltpu.VMEM((NCHUNK, C), jnp.int32),
        pltpu.VMEM((NCHUNK, C), jnp.float32),
        pltpu.VMEM((NBUF, C, HW), jnp.int32),
        pltpu.VMEM((C, HALF), jnp.float32),
        pltpu.VMEM((C, HW), jnp.int32),
        pltpu.VMEM((RPT,), jnp.float32),
        pltpu.VMEM((C,), jnp.float32),
        pltpu.VMEM((32,), jnp.float32),
        pltpu.VMEM((C,), jnp.float32),
        pltpu.VMEM((C,), jnp.float32),
        pltpu.SemaphoreType.DMA((NBUF,)),
        pltpu.VMEM_SHARED((NP, HALF), jnp.float32),
        pltpu.VMEM_SHARED((NP,), jnp.float32),
    ],
)(_body)


def kernel(x, edge_index, edge_attr, pe_alphas):
    row = edge_index[0]
    col = edge_index[1]
    pad = EPT - E // NS
    rp = jnp.concatenate(
        [row.reshape(NS, E // NS), jnp.full((NS, pad), N, jnp.int32)],
        axis=1).reshape(NS * NCHUNK, C)
    cp = jnp.concatenate(
        [col.reshape(NS, E // NS), jnp.zeros((NS, pad), jnp.int32)],
        axis=1).reshape(NS * NCHUNK, C)
    ap = jnp.concatenate(
        [edge_attr.reshape(NS, E // NS), jnp.zeros((NS, pad), jnp.float32)],
        axis=1).reshape(NS * NCHUNK, C)
    xp = jnp.pad(x, ((0, NP - N), (0, 0)))
    xr = xp.reshape(NP, NC, HALF).transpose(1, 0, 2).reshape(NC * NP, HALF)
    pe = jnp.pad(pe_alphas.astype(jnp.float32), (0, 32 - DEPTH))
    zz = jnp.zeros((C, HALF), jnp.float32)

    out, _ = _sc_call(xr, rp, cp, ap, pe, zz)
    out = out.reshape(NC, DEPTH + 1, NP, HALF)[:, :, :N, :]
    out = out.transpose(2, 1, 0, 3).reshape(N, DEPTH + 1, D)
    return out


# speedup vs baseline: 1.9296x; 1.1217x over previous
import functools

import jax
import jax.numpy as jnp
from jax import lax
from jax.experimental import pallas as pl
from jax.experimental.pallas import tpu as pltpu
from jax.experimental.pallas import tpu_sc as plsc

N = 10000
E = 320000
D = 128
DEPTH = 10

NC = 2
NS = 16
HALF = D // NC
HW = HALF // 2
NP = 10240
RPT = NP // NS
C = 128
EPT = 20480
NCHUNK = EPT // C
NWB = RPT // C
OUTROWS = NC * (DEPTH + 1) * NP
NBUF = 2


def _body(xr, rowh, colh, attrh, peh, zz, out, hb,
          row_v, col_v, val_v, gb16, prod, hbuf, dinv_l, ones_b, alph,
          tmp_r, tmp_c, gsems, acc, degacc):
    c = lax.axis_index("c")
    s = lax.axis_index("s")
    base0 = c * ((DEPTH + 1) * NP)
    hbase = c * NP
    e0 = s * NCHUNK
    r0 = s * RPT

    pltpu.sync_copy(rowh.at[pl.ds(e0, NCHUNK)], row_v)
    pltpu.sync_copy(colh.at[pl.ds(e0, NCHUNK)], col_v)
    pltpu.sync_copy(attrh.at[pl.ds(e0, NCHUNK)], val_v)
    pltpu.sync_copy(peh, alph)

    for g in range(2):
        sl = pl.ds(16 * g, 16)
        pe = alph[sl]
        alph[sl] = 1.0 - 2.0 / (jnp.exp(pe * 2.0) + 1.0)

    one16 = jnp.ones((16,), jnp.float32)
    for g in range(C // 16):
        ones_b[pl.ds(16 * g, 16)] = one16

    for k in range(RPT // C):
        pltpu.sync_copy(zz, acc.at[pl.ds(r0 + k * C, C)])
    for k in range(RPT // HALF):
        pltpu.sync_copy(zz.at[0], degacc.at[pl.ds(r0 + k * HALF, HALF)])
    plsc.subcore_barrier()

    def deg_chunk(j, carry):
        pltpu.sync_copy(ones_b, degacc.at[row_v.at[j]], add=True)
        return carry
    lax.fori_loop(0, NCHUNK, deg_chunk, 0)
    plsc.subcore_barrier()

    pltpu.sync_copy(degacc.at[pl.ds(r0, RPT)], dinv_l)
    for g in range(RPT // 16):
        sl = pl.ds(16 * g, 16)
        d = dinv_l[sl]
        d = jnp.where(d < 0.5, d + 1.0, d)
        i = lax.bitcast_convert_type(d, jnp.int32)
        i = 0x5F3759DF - lax.shift_right_logical(i, 1)
        y = lax.bitcast_convert_type(i, jnp.float32)
        for _ in range(3):
            y = y * (1.5 - 0.5 * d * y * y)
        dinv_l[sl] = y
    pltpu.sync_copy(dinv_l, degacc.at[pl.ds(r0, RPT)])
    plsc.subcore_barrier()

    b016 = jnp.full((16,), hbase, jnp.int32)

    def val_chunk(j, carry):
        pltpu.sync_copy(degacc.at[row_v.at[j]], tmp_r)
        pltpu.sync_copy(degacc.at[col_v.at[j]], tmp_c)
        for g in range(C // 16):
            sl = pl.ds(16 * g, 16)
            val_v[j, sl] = tmp_r[sl] * val_v[j, sl] * tmp_c[sl]
            col_v[j, sl] = col_v[j, sl] + b016
        return carry
    lax.fori_loop(0, NCHUNK, val_chunk, 0)

    def pack_rows(sbuf, dst):
        def prow(r, cy):
            for g in range(HALF // 32):
                a = sbuf[r, pl.ds(32 * g, 16)]
                b = sbuf[r, pl.ds(32 * g + 16, 16)]
                ta = lax.bitcast_convert_type(a, jnp.int32) + 0x8000
                tb = lax.bitcast_convert_type(b, jnp.int32) + 0x8000
                ra = lax.shift_right_logical(ta, 16)
                rb = lax.bitwise_and(tb, -65536)
                dst[r, pl.ds(16 * g, 16)] = lax.bitwise_or(ra, rb)
            return cy
        lax.fori_loop(0, C, prow, 0)

    for k in range(NWB):
        pltpu.sync_copy(xr.at[pl.ds(c * NP + r0 + k * C, C)], prod)
        pltpu.sync_copy(prod, out.at[pl.ds(base0 + r0 + k * C, C)])
        pack_rows(prod, hbuf)
        pltpu.sync_copy(hbuf, hb.at[pl.ds(hbase + r0 + k * C, C)])
    plsc.subcore_barrier()

    def depth_body(l, carry):
        pltpu.async_copy(hb.at[col_v.at[0]], gb16.at[0], gsems.at[0])

        def chunk(j, cy):
            for b in range(NBUF):
                jj = j * NBUF + b
                gb = gb16.at[b]
                bp = (b + 1) % NBUF
                @pl.when(jj + 1 < NCHUNK)
                def _():
                    pltpu.async_copy(hb.at[col_v.at[jj + 1]],
                                     gb16.at[bp], gsems.at[bp])

                pltpu.make_async_copy(hb.at[col_v.at[jj]], gb,
                                      gsems.at[b]).wait()

                for g in range(C // 16):
                    v16 = val_v[jj, pl.ds(16 * g, 16)]
                    for i in range(16):
                        e = 16 * g + i
                        vv = jnp.full((16,), v16[i], jnp.float32)
                        for f in range(HALF // 32):
                            w = gb[e, pl.ds(16 * f, 16)]
                            lo = lax.bitcast_convert_type(
                                lax.shift_left(w, 16), jnp.float32)
                            hi = lax.bitcast_convert_type(
                                lax.bitwise_and(w, -65536), jnp.float32)
                            prod[e, pl.ds(32 * f, 16)] = lo * vv
                            prod[e, pl.ds(32 * f + 16, 16)] = hi * vv
                pltpu.sync_copy(prod, acc.at[row_v.at[jj]], add=True)
            return cy
        lax.fori_loop(0, NCHUNK // NBUF, chunk, 0)
        plsc.subcore_barrier()

        a16 = alph[pl.ds(l - 1, 16)]
        av = jnp.full((16,), a16[0], jnp.float32)

        def wb_chunk(k, cy):
            rr = r0 + k * C
            pltpu.sync_copy(acc.at[pl.ds(rr, C)], prod)
            pltpu.sync_copy(zz, acc.at[pl.ds(rr, C)])

            def wrow(r, cy2):
                for f in range(HALF // 16):
                    sl = pl.ds(16 * f, 16)
                    prod[r, sl] = prod[r, sl] * av
                return cy2
            lax.fori_loop(0, C, wrow, 0)
            pltpu.sync_copy(prod, out.at[pl.ds(base0 + l * NP + rr, C)])
            pack_rows(prod, hbuf)
            pltpu.sync_copy(hbuf, hb.at[pl.ds(hbase + rr, C)])
            return cy
        lax.fori_loop(0, NWB, wb_chunk, 0)
        plsc.subcore_barrier()
        return carry
    lax.fori_loop(1, DEPTH + 1, depth_body, 0)


_mesh = plsc.VectorSubcoreMesh(core_axis_name="c", subcore_axis_name="s")

_sc_call = functools.partial(
    pl.kernel,
    out_type=(jax.ShapeDtypeStruct((OUTROWS, HALF), jnp.float32),
              jax.ShapeDtypeStruct((NC * NP, HW), jnp.int32)),
    mesh=_mesh,
    compiler_params=pltpu.CompilerParams(use_tc_tiling_on_sc=False),
    scratch_types=[
        pltpu.VMEM((NCHUNK, C), jnp.int32),
        pltpu.VMEM((NCHUNK, C), jnp.int32),
        pltpu.VMEM((NCHUNK, C), jnp.float32),
        pltpu.VMEM((NBUF, C, HW), jnp.int32),
        pltpu.VMEM((C, HALF), jnp.float32),
        pltpu.VMEM((C, HW), jnp.int32),
        pltpu.VMEM((RPT,), jnp.float32),
        pltpu.VMEM((C,), jnp.float32),
        pltpu.VMEM((32,), jnp.float32),
        pltpu.VMEM((C,), jnp.float32),
        pltpu.VMEM((C,), jnp.float32),
        pltpu.SemaphoreType.DMA((NBUF,)),
        pltpu.VMEM_SHARED((NP, HALF), jnp.float32),
        pltpu.VMEM_SHARED((NP,), jnp.float32),
    ],
)(_body)


def kernel(x, edge_index, edge_attr, pe_alphas):
    row = edge_index[0]
    col = edge_index[1]
    pad = EPT - E // NS
    rp = jnp.concatenate(
        [row.reshape(NS, E // NS), jnp.full((NS, pad), N, jnp.int32)],
        axis=1).reshape(NS * NCHUNK, C)
    cp = jnp.concatenate(
        [col.reshape(NS, E // NS), jnp.zeros((NS, pad), jnp.int32)],
        axis=1).reshape(NS * NCHUNK, C)
    ap = jnp.concatenate(
        [edge_attr.reshape(NS, E // NS), jnp.zeros((NS, pad), jnp.float32)],
        axis=1).reshape(NS * NCHUNK, C)
    xp = jnp.pad(x, ((0, NP - N), (0, 0)))
    xr = xp.reshape(NP, NC, HALF).transpose(1, 0, 2).reshape(NC * NP, HALF)
    pe = jnp.pad(pe_alphas.astype(jnp.float32), (0, 32 - DEPTH))
    zz = jnp.zeros((C, HALF), jnp.float32)

    out, _ = _sc_call(xr, rp, cp, ap, pe, zz)
    out = out.reshape(NC, DEPTH + 1, NP, HALF)[:, :, :N, :]
    out = out.transpose(2, 1, 0, 3).reshape(N, DEPTH + 1, D)
    return out
